# baseline (speedup 1.0000x reference)
import functools

import jax
import jax.numpy as jnp
from jax import lax
from jax.experimental import pallas as pl
from jax.experimental.pallas import tpu as pltpu
from jax.experimental.pallas import tpu_sc as plsc

_VOCAB = 100000
_D = 64
_B = 4096
_L = 200

_NC = 2
_NS = 16
_NW = _NC * _NS
_RPW = _B // _NW

_LA = 104
_LB = _L - _LA

_VB = 8192


def _make_sc_bag_body(npos, off):
    def body(idx_hbm, emb_hbm, out_hbm,
             idx_v, a0_v, a1_v, a2_v, a3_v, b0_v, b1_v, b2_v, b3_v, acc_v,
             sem0, sem1):
        c = lax.axis_index("c")
        s = lax.axis_index("s")
        wid = s * _NC + c
        base = wid * _RPW

        pltpu.sync_copy(idx_hbm.at[pl.ds(base, _RPW), pl.ds(off, npos)], idx_v)

        bufs_a = (a0_v, a1_v, a2_v, a3_v)
        bufs_b = (b0_v, b1_v, b2_v, b3_v)

        def fire_quad(j, bufs, sem):
            for i in range(4):
                pltpu.async_copy(emb_hbm.at[idx_v.at[4 * j + i]], bufs[i], sem)

        def drain_quad(bufs, sem):
            for i in range(4):
                pltpu.make_async_copy(emb_hbm.at[pl.ds(0, npos)], bufs[i], sem).wait()

        def init4(bufs):
            b0, b1, b2, b3 = bufs

            @plsc.parallel_loop(0, npos, unroll=4)
            def _(l):
                for cc in range(_D // 16):
                    sl = pl.ds(cc * 16, 16)
                    acc_v[l, sl] = (b0[l, sl] + b1[l, sl]) + (b2[l, sl] + b3[l, sl])

        def accum4(bufs):
            b0, b1, b2, b3 = bufs

            @plsc.parallel_loop(0, npos, unroll=4)
            def _(l):
                for cc in range(_D // 16):
                    sl = pl.ds(cc * 16, 16)
                    plsc.addupdate(
                        acc_v.at[l, sl],
                        (b0[l, sl] + b1[l, sl]) + (b2[l, sl] + b3[l, sl]),
                    )

        nq = _RPW // 4

        fire_quad(0, bufs_a, sem0)
        fire_quad(1, bufs_b, sem1)
        drain_quad(bufs_a, sem0)
        init4(bufs_a)
        fire_quad(2, bufs_a, sem0)

        def loop(j, _):
            drain_quad(bufs_b, sem1)
            accum4(bufs_b)
            fire_quad(2 * j + 1, bufs_b, sem1)
            drain_quad(bufs_a, sem0)
            accum4(bufs_a)

            @pl.when(j < nq // 2 - 1)
            def _():
                fire_quad(2 * j + 2, bufs_a, sem0)

            return 0

        lax.fori_loop(1, nq // 2, loop, 0)

        drain_quad(bufs_b, sem1)
        accum4(bufs_b)

        pltpu.sync_copy(acc_v, out_hbm.at[wid])

    return body


@functools.cache
def _sc_bag(npos, off):
    buf = pltpu.VMEM((npos, _D), jnp.float32)
    return pl.kernel(
        _make_sc_bag_body(npos, off),
        out_type=jax.ShapeDtypeStruct((_NW, npos, _D), jnp.float32),
        mesh=plsc.VectorSubcoreMesh(core_axis_name="c", subcore_axis_name="s"),
        scratch_types=[
            pltpu.VMEM((_RPW, npos), jnp.int32),
            buf, buf, buf, buf, buf, buf, buf, buf,
            pltpu.VMEM((npos, _D), jnp.float32),
            pltpu.SemaphoreType.DMA,
            pltpu.SemaphoreType.DMA,
        ],
        compiler_params=pltpu.CompilerParams(use_tc_tiling_on_sc=False),
    )


def _tc_proj_a_body(p_ref, w_ref, b_ref, o_ref, s_ref):
    @pl.when(pl.program_id(0) == 0)
    def _():
        s_ref[...] = jnp.sum(p_ref[...], axis=0) * (1.0 / _B)

    o_ref[...] = (
        lax.dot_general(
            s_ref[...], w_ref[...], (((1,), (1,)), ((), ())),
            preferred_element_type=jnp.float32,
        )
        + b_ref[...]
    )


def _tc_proj_b_body(p_ref, w_ref, b_ref, prev_ref, o_ref, s_ref):
    del prev_ref

    @pl.when(pl.program_id(0) == 0)
    def _():
        s_ref[pl.ds(0, _LB), :] = jnp.sum(p_ref[...], axis=0) * (1.0 / _B)
        s_ref[pl.ds(_LB, _LA - _LB), :] = jnp.zeros((_LA - _LB, _D), jnp.float32)

    o_ref[...] = (
        lax.dot_general(
            s_ref[...], w_ref[...], (((1,), (1,)), ((), ())),
            preferred_element_type=jnp.float32,
        )
        + b_ref[...]
    )


def _tc_proj_a(partials, W, b2):
    grid = (pl.cdiv(_VOCAB, _VB),)
    return pl.pallas_call(
        _tc_proj_a_body,
        grid=grid,
        in_specs=[
            pl.BlockSpec((_NW, _LA, _D), lambda i: (0, 0, 0)),
            pl.BlockSpec((_VB, _D), lambda i: (i, 0)),
            pl.BlockSpec((1, _VB), lambda i: (0, i)),
        ],
        out_specs=pl.BlockSpec((_LA, _VB), lambda i: (0, i)),
        out_shape=jax.ShapeDtypeStruct((_L, _VOCAB), jnp.float32),
        scratch_shapes=[pltpu.VMEM((_LA, _D), jnp.float32)],
    )(partials, W, b2)


def _tc_proj_b(partials, W, b2, prev):
    grid = (pl.cdiv(_VOCAB, _VB),)
    return pl.pallas_call(
        _tc_proj_b_body,
        grid=grid,
        in_specs=[
            pl.BlockSpec((_NW, _LB, _D), lambda i: (0, 0, 0)),
            pl.BlockSpec((_VB, _D), lambda i: (i, 0)),
            pl.BlockSpec((1, _VB), lambda i: (0, i)),
            pl.BlockSpec(memory_space=pl.ANY),
        ],
        out_specs=pl.BlockSpec((_LA, _VB), lambda i: (1, i)),
        out_shape=jax.ShapeDtypeStruct((_L, _VOCAB), jnp.float32),
        scratch_shapes=[pltpu.VMEM((_LA, _D), jnp.float32)],
        input_output_aliases={3: 0},
    )(partials, W, b2, prev)


@jax.jit
def kernel(inputs, emb, W, b):
    idx = inputs.astype(jnp.int32)
    b2 = b.reshape(1, _VOCAB)
    partials_a = _sc_bag(_LA, 0)(idx, emb)
    out_a = _tc_proj_a(partials_a, W, b2)
    partials_b = _sc_bag(_LB, _LA)(idx, emb)
    return _tc_proj_b(partials_b, W, b2, out_a)


# speedup vs baseline: 1.0014x; 1.0014x over previous
import functools

import jax
import jax.numpy as jnp
from jax import lax
from jax.experimental import pallas as pl
from jax.experimental.pallas import tpu as pltpu
from jax.experimental.pallas import tpu_sc as plsc

_VOCAB = 100000
_D = 64
_B = 4096
_L = 200

_NC = 2
_NS = 16
_NW = _NC * _NS
_RPW = _B // _NW

_LA = 104
_LB = _L - _LA

_VB = 8192


def _make_sc_bag_body(npos, off):
    def body(idx_hbm, emb_hbm, out_hbm,
             idx_v, a0_v, a1_v, a2_v, a3_v, b0_v, b1_v, b2_v, b3_v, acc_v,
             sem0, sem1):
        c = lax.axis_index("c")
        s = lax.axis_index("s")
        wid = s * _NC + c
        base = wid * _RPW

        pltpu.sync_copy(idx_hbm.at[pl.ds(base, _RPW), pl.ds(off, npos)], idx_v)

        bufs_a = (a0_v, a1_v, a2_v, a3_v)
        bufs_b = (b0_v, b1_v, b2_v, b3_v)

        def fire_quad(j, bufs, sem):
            for i in range(4):
                pltpu.async_copy(emb_hbm.at[idx_v.at[4 * j + i]], bufs[i], sem)

        def drain_quad(bufs, sem):
            for i in range(4):
                pltpu.make_async_copy(emb_hbm.at[pl.ds(0, npos)], bufs[i], sem).wait()

        def init4(bufs):
            b0, b1, b2, b3 = bufs

            @plsc.parallel_loop(0, npos, unroll=4)
            def _(l):
                for cc in range(_D // 16):
                    sl = pl.ds(cc * 16, 16)
                    acc_v[l, sl] = (b0[l, sl] + b1[l, sl]) + (b2[l, sl] + b3[l, sl])

        def accum4(bufs):
            b0, b1, b2, b3 = bufs

            @plsc.parallel_loop(0, npos, unroll=4)
            def _(l):
                for cc in range(_D // 16):
                    sl = pl.ds(cc * 16, 16)
                    plsc.addupdate(
                        acc_v.at[l, sl],
                        (b0[l, sl] + b1[l, sl]) + (b2[l, sl] + b3[l, sl]),
                    )

        nq = _RPW // 4

        fire_quad(0, bufs_a, sem0)
        fire_quad(1, bufs_b, sem1)
        drain_quad(bufs_a, sem0)
        init4(bufs_a)
        fire_quad(2, bufs_a, sem0)

        def loop(j, _):
            drain_quad(bufs_b, sem1)
            accum4(bufs_b)
            fire_quad(2 * j + 1, bufs_b, sem1)
            drain_quad(bufs_a, sem0)
            accum4(bufs_a)

            @pl.when(j < nq // 2 - 1)
            def _():
                fire_quad(2 * j + 2, bufs_a, sem0)

            return 0

        lax.fori_loop(1, nq // 2, loop, 0)

        drain_quad(bufs_b, sem1)
        accum4(bufs_b)

        pltpu.sync_copy(acc_v, out_hbm.at[wid])

    return body


@functools.cache
def _sc_bag(npos, off):
    buf = pltpu.VMEM((npos, _D), jnp.float32)
    return pl.kernel(
        _make_sc_bag_body(npos, off),
        out_type=jax.ShapeDtypeStruct((_NW, npos, _D), jnp.float32),
        mesh=plsc.VectorSubcoreMesh(core_axis_name="c", subcore_axis_name="s"),
        scratch_types=[
            pltpu.VMEM((_RPW, npos), jnp.int32),
            buf, buf, buf, buf, buf, buf, buf, buf,
            pltpu.VMEM((npos, _D), jnp.float32),
            pltpu.SemaphoreType.DMA,
            pltpu.SemaphoreType.DMA,
        ],
        compiler_params=pltpu.CompilerParams(use_tc_tiling_on_sc=False),
    )


def _tc_proj_a_body(p_ref, w_ref, b_ref, o_ref, s_ref):
    @pl.when(pl.program_id(0) == 0)
    def _():
        s_ref[...] = jnp.sum(p_ref[...], axis=0) * (1.0 / _B)

    o_ref[...] = (
        lax.dot_general(
            s_ref[...], w_ref[...], (((1,), (1,)), ((), ())),
            preferred_element_type=jnp.float32,
        )
        + b_ref[...]
    )


def _tc_proj_b_body(p_ref, w_ref, b_ref, prev_ref, o_ref, s_ref):
    del prev_ref

    @pl.when(pl.program_id(0) == 0)
    def _():
        s_ref[pl.ds(0, _LB), :] = jnp.sum(p_ref[...], axis=0) * (1.0 / _B)
        s_ref[pl.ds(_LB, _LA - _LB), :] = jnp.zeros((_LA - _LB, _D), jnp.float32)

    o_ref[...] = (
        lax.dot_general(
            s_ref[...], w_ref[...], (((1,), (1,)), ((), ())),
            preferred_element_type=jnp.float32,
        )
        + b_ref[...]
    )


def _tc_proj_a(partials, W, b2):
    grid = (pl.cdiv(_VOCAB, _VB),)
    return pl.pallas_call(
        _tc_proj_a_body,
        grid=grid,
        in_specs=[
            pl.BlockSpec((_NW, _LA, _D), lambda i: (0, 0, 0)),
            pl.BlockSpec((_VB, _D), lambda i: (i, 0)),
            pl.BlockSpec((1, _VB), lambda i: (0, i)),
        ],
        out_specs=pl.BlockSpec((_LA, _VB), lambda i: (0, i)),
        out_shape=jax.ShapeDtypeStruct((_L, _VOCAB), jnp.float32),
        scratch_shapes=[pltpu.VMEM((_LA, _D), jnp.float32)],
    )(partials, W, b2)


def _tc_proj_b(partials, W, b2, prev):
    grid = (pl.cdiv(_VOCAB, _VB),)
    return pl.pallas_call(
        _tc_proj_b_body,
        grid=grid,
        in_specs=[
            pl.BlockSpec((_NW, _LB, _D), lambda i: (0, 0, 0)),
            pl.BlockSpec((_VB, _D), lambda i: (i, 0)),
            pl.BlockSpec((1, _VB), lambda i: (0, i)),
            pl.BlockSpec(memory_space=pl.ANY),
        ],
        out_specs=pl.BlockSpec((_LA, _VB), lambda i: (1, i)),
        out_shape=jax.ShapeDtypeStruct((_L, _VOCAB), jnp.float32),
        scratch_shapes=[pltpu.VMEM((_LA, _D), jnp.float32)],
        input_output_aliases={3: 0},
    )(partials, W, b2, prev)


@jax.jit
def kernel(inputs, emb, W, b):
    idx = inputs.astype(jnp.int32)
    b2 = b.reshape(1, _VOCAB)
    partials_a = _sc_bag(_LA, 0)(idx, emb)
    partials_b = _sc_bag(_LB, _LA)(idx, emb)
    out_a = _tc_proj_a(partials_a, W, b2)
    return _tc_proj_b(partials_b, W, b2, out_a)


# speedup vs baseline: 1.0528x; 1.0512x over previous
import functools

import jax
import jax.numpy as jnp
from jax import lax
from jax.experimental import pallas as pl
from jax.experimental.pallas import tpu as pltpu
from jax.experimental.pallas import tpu_sc as plsc

_VOCAB = 100000
_D = 64
_B = 4096
_L = 200

_NC = 2
_NS = 16
_NW = _NC * _NS
_RPW = _B // _NW

_VB = 12800


def _sc_bag_body(
    idx_hbm, emb_hbm, out_hbm,
    idx_v, a0_v, a1_v, a2_v, a3_v, b0_v, b1_v, b2_v, b3_v, acc_v, sem0, sem1,
):
    c = lax.axis_index("c")
    s = lax.axis_index("s")
    wid = s * _NC + c
    base = wid * _RPW
    half = _L // 2

    pltpu.sync_copy(idx_hbm.at[pl.ds(base, _RPW)], idx_v)

    bufs_a = (a0_v, a1_v, a2_v, a3_v)
    bufs_b = (b0_v, b1_v, b2_v, b3_v)

    def fire_quad(j, h, bufs, sem):
        for i in range(4):
            pltpu.async_copy(emb_hbm.at[idx_v.at[4 * j + i, h]], bufs[i], sem)

    def drain_quad(bufs, sem):
        for i in range(4):
            pltpu.make_async_copy(emb_hbm.at[pl.ds(0, half)], bufs[i], sem).wait()

    def init4(bufs, h):
        b0, b1, b2, b3 = bufs

        @plsc.parallel_loop(0, half, unroll=4)
        def _(l):
            for cc in range(_D // 16):
                sl = pl.ds(cc * 16, 16)
                acc_v[h * half + l, sl] = (b0[l, sl] + b1[l, sl]) + (
                    b2[l, sl] + b3[l, sl]
                )

    def accum4(bufs, h):
        b0, b1, b2, b3 = bufs

        @plsc.parallel_loop(0, half, unroll=4)
        def _(l):
            for cc in range(_D // 16):
                sl = pl.ds(cc * 16, 16)
                plsc.addupdate(
                    acc_v.at[h * half + l, sl],
                    (b0[l, sl] + b1[l, sl]) + (b2[l, sl] + b3[l, sl]),
                )

    nq = _RPW // 4

    fire_quad(0, 0, bufs_a, sem0)
    fire_quad(0, 1, bufs_b, sem1)
    drain_quad(bufs_a, sem0)
    init4(bufs_a, 0)
    fire_quad(1, 0, bufs_a, sem0)
    drain_quad(bufs_b, sem1)
    init4(bufs_b, 1)
    fire_quad(1, 1, bufs_b, sem1)

    def body(j, _):
        drain_quad(bufs_a, sem0)
        accum4(bufs_a, 0)

        @pl.when(j < nq - 1)
        def _():
            fire_quad(j + 1, 0, bufs_a, sem0)

        drain_quad(bufs_b, sem1)
        accum4(bufs_b, 1)

        @pl.when(j < nq - 1)
        def _():
            fire_quad(j + 1, 1, bufs_b, sem1)

        return 0

    lax.fori_loop(1, nq, body, 0)

    pltpu.sync_copy(acc_v, out_hbm.at[wid])


@functools.cache
def _sc_bag():
    buf = pltpu.VMEM((_L // 2, _D), jnp.float32)
    return pl.kernel(
        _sc_bag_body,
        out_type=jax.ShapeDtypeStruct((_NW, _L, _D), jnp.float32),
        mesh=plsc.VectorSubcoreMesh(core_axis_name="c", subcore_axis_name="s"),
        scratch_types=[
            pltpu.VMEM((_RPW, 2, _L // 2), jnp.int32),
            buf, buf, buf, buf, buf, buf, buf, buf,
            pltpu.VMEM((_L, _D), jnp.float32),
            pltpu.SemaphoreType.DMA,
            pltpu.SemaphoreType.DMA,
        ],
        compiler_params=pltpu.CompilerParams(use_tc_tiling_on_sc=False),
    )


def _tc_proj_body(p_ref, w_ref, b_ref, o_ref, s_ref):
    @pl.when(pl.program_id(0) == 0)
    def _():
        s_ref[...] = jnp.sum(p_ref[...], axis=0) * (1.0 / _B)

    o_ref[...] = (
        lax.dot_general(
            s_ref[...], w_ref[...], (((1,), (1,)), ((), ())),
            preferred_element_type=jnp.float32,
        )
        + b_ref[...]
    )


def _tc_proj(partials, W, b2):
    grid = (pl.cdiv(_VOCAB, _VB),)
    return pl.pallas_call(
        _tc_proj_body,
        grid=grid,
        in_specs=[
            pl.BlockSpec((_NW, _L, _D), lambda i: (0, 0, 0)),
            pl.BlockSpec((_VB, _D), lambda i: (i, 0)),
            pl.BlockSpec((1, _VB), lambda i: (0, i)),
        ],
        out_specs=pl.BlockSpec((_L, _VB), lambda i: (0, i)),
        out_shape=jax.ShapeDtypeStruct((_L, _VOCAB), jnp.float32),
        scratch_shapes=[pltpu.VMEM((_L, _D), jnp.float32)],
    )(partials, W, b2)


@jax.jit
def kernel(inputs, emb, W, b):
    idx = inputs.astype(jnp.int32).reshape(_B, 2, _L // 2)
    partials = _sc_bag()(idx, emb)
    return _tc_proj(partials, W, b.reshape(1, _VOCAB))
